# baseline (speedup 1.0000x reference)
import jax
import jax.numpy as jnp
from jax.experimental import pallas as pl

_B, _N, _T, _C, _H = 4, 256, 12, 64, 4
_D = _C // _H


def _attn_body(x_ref, adj_ref, w_ref, a_ref, cw_ref, cb_ref, o_ref):
    xb = x_ref[0]
    h = jnp.dot(xb, w_ref[...], preferred_element_type=jnp.float32)
    ed = jnp.dot(h, a_ref[...], preferred_element_type=jnp.float32)
    adj = adj_ref[...]
    outs = []
    for hh in range(_H):
        src = ed[:, hh][:, None]
        dst = ed[:, _H + hh][None, :]
        e = src + dst
        e = jnp.where(e >= 0, e, 0.2 * e)
        e = jnp.where(adj > 0.5, e, jnp.float32(-1e9))
        m = jnp.max(e, axis=1, keepdims=True)
        p = jnp.exp(e - m)
        alpha = p / jnp.sum(p, axis=1, keepdims=True)
        outs.append(jnp.dot(alpha, h[:, hh * _D:(hh + 1) * _D],
                            preferred_element_type=jnp.float32))
    out = jnp.concatenate(outs, axis=1)
    y = jnp.dot(out, cw_ref[...], preferred_element_type=jnp.float32)
    y = y + cb_ref[...]
    o_ref[0] = jnp.maximum(y, 0.0)


def kernel(x, adj, W, a_src, a_dst, conv_w, conv_b):
    xt = jnp.transpose(x, (0, 2, 1, 3)).reshape(_B * _T, _N, _C)

    eye_h = jnp.eye(_H, dtype=jnp.float32)
    blk_src = jnp.einsum('hd,hg->hdg', a_src, eye_h).reshape(_C, _H)
    blk_dst = jnp.einsum('hd,hg->hdg', a_dst, eye_h).reshape(_C, _H)
    A = jnp.concatenate([blk_src, blk_dst], axis=1)

    cb2 = conv_b.reshape(1, _C)
    cwt = conv_w.T

    grid = (_B * _T,)
    y = pl.pallas_call(
        _attn_body,
        grid=grid,
        in_specs=[
            pl.BlockSpec((1, _N, _C), lambda i: (i, 0, 0)),
            pl.BlockSpec((_N, _N), lambda i: (0, 0)),
            pl.BlockSpec((_C, _C), lambda i: (0, 0)),
            pl.BlockSpec((_C, 2 * _H), lambda i: (0, 0)),
            pl.BlockSpec((_C, _C), lambda i: (0, 0)),
            pl.BlockSpec((1, _C), lambda i: (0, 0)),
        ],
        out_specs=pl.BlockSpec((1, _N, _C), lambda i: (i, 0, 0)),
        out_shape=jax.ShapeDtypeStruct((_B * _T, _N, _C), jnp.float32),
    )(xt, adj, W, A, cwt, cb2)

    return jnp.transpose(y.reshape(_B, _T, _N, _C), (0, 2, 1, 3))


# speedup vs baseline: 2.3378x; 2.3378x over previous
import jax
import jax.numpy as jnp
import numpy as np
from jax.experimental import pallas as pl

_B, _N, _T, _C, _H = 4, 256, 12, 64, 4
_D = _C // _H


def _attn_body(x_ref, adjt_ref, wt_ref, a_ref, cw_ref, cb_ref, o_ref):
    xbT = x_ref[0]
    hT = jnp.dot(wt_ref[...], xbT, preferred_element_type=jnp.float32)
    edr = jnp.dot(a_ref[...], hT, preferred_element_type=jnp.float32)
    edc = edr.T
    adjT = adjt_ref[...]
    dmax = jnp.max(edr[_H:2 * _H, :], axis=1, keepdims=True)
    tm = edr[:_H, :] + dmax
    m_rows = jnp.maximum(tm, 0.2 * tm)

    ones_row = jnp.ones((1, _N), dtype=jnp.float32)
    outs = []
    denoms = []
    for hh in range(_H):
        t = edc[:, _H + hh][:, None] + edr[hh][None, :]
        u = jnp.maximum(t, 0.2 * t)
        pT = jnp.exp2(u - m_rows[hh][None, :]) * adjT
        hs = jnp.concatenate([hT[hh * _D:(hh + 1) * _D, :], ones_row], axis=0)
        agg = jnp.dot(hs, pT, preferred_element_type=jnp.float32)
        outs.append(agg[:_D, :])
        denoms.append(agg[_D:, :])
    recips = [1.0 / d for d in denoms]
    outT = jnp.concatenate([o * r for o, r in zip(outs, recips)], axis=0)
    yT = jnp.dot(cw_ref[...], outT, preferred_element_type=jnp.float32)
    yT = yT + cb_ref[...]
    o_ref[0] = jnp.maximum(yT, 0.0)


def kernel(x, adj, W, a_src, a_dst, conv_w, conv_b):
    xt = jnp.transpose(x, (0, 2, 3, 1)).reshape(_B * _T, _C, _N)
    adjt = adj.T

    log2e = jnp.float32(np.log2(np.e))
    eye_h = jnp.eye(_H, dtype=jnp.float32)
    blk_src = jnp.einsum('hd,hg->ghd', a_src, eye_h).reshape(_H, _C)
    blk_dst = jnp.einsum('hd,hg->ghd', a_dst, eye_h).reshape(_H, _C)
    A = jnp.concatenate([blk_src, blk_dst], axis=0) * log2e

    cb_col = conv_b.reshape(_C, 1)

    y = pl.pallas_call(
        _attn_body,
        grid=(_B * _T,),
        in_specs=[
            pl.BlockSpec((1, _C, _N), lambda i: (i, 0, 0)),
            pl.BlockSpec((_N, _N), lambda i: (0, 0)),
            pl.BlockSpec((_C, _C), lambda i: (0, 0)),
            pl.BlockSpec((2 * _H, _C), lambda i: (0, 0)),
            pl.BlockSpec((_C, _C), lambda i: (0, 0)),
            pl.BlockSpec((_C, 1), lambda i: (0, 0)),
        ],
        out_specs=pl.BlockSpec((1, _C, _N), lambda i: (i, 0, 0)),
        out_shape=jax.ShapeDtypeStruct((_B * _T, _C, _N), jnp.float32),
    )(xt, adjt, W.T, A, conv_w, cb_col)

    return jnp.transpose(y.reshape(_B, _T, _C, _N), (0, 3, 1, 2))


# speedup vs baseline: 2.8566x; 1.2219x over previous
import jax
import jax.numpy as jnp
import numpy as np
from jax.experimental import pallas as pl

_B, _N, _T, _C, _H = 4, 256, 12, 64, 4
_D = _C // _H


_S = 4


def _attn_body(x_ref, adjt_ref, wt_ref, a_ref, cw_ref, cb_ref, o_ref):
    adjT = adjt_ref[...]
    ones_row = jnp.ones((1, _N), dtype=jnp.float32)
    for s in range(_S):
        xbT = x_ref[s]
        hT = jnp.dot(wt_ref[...], xbT, preferred_element_type=jnp.float32)
        edr = jnp.dot(a_ref[...], hT, preferred_element_type=jnp.float32)
        edc = edr.T
        dmax = jnp.max(edr[_H:2 * _H, :], axis=1, keepdims=True)
        tm = edr[:_H, :] + dmax
        m_rows = jnp.maximum(tm, 0.2 * tm)

        outs = []
        denoms = []
        for hh in range(_H):
            t = edc[:, _H + hh][:, None] + edr[hh][None, :]
            u = jnp.maximum(t, 0.2 * t)
            pT = jnp.exp2(u - m_rows[hh][None, :]) * adjT
            hs = jnp.concatenate([hT[hh * _D:(hh + 1) * _D, :], ones_row],
                                 axis=0)
            agg = jnp.dot(hs, pT, preferred_element_type=jnp.float32)
            outs.append(agg[:_D, :])
            denoms.append(agg[_D:, :])
        recips = [1.0 / d for d in denoms]
        outT = jnp.concatenate([o * r for o, r in zip(outs, recips)], axis=0)
        yT = jnp.dot(cw_ref[...], outT, preferred_element_type=jnp.float32)
        yT = yT + cb_ref[...]
        o_ref[s] = jnp.maximum(yT, 0.0)


def kernel(x, adj, W, a_src, a_dst, conv_w, conv_b):
    xt = jnp.transpose(x, (0, 2, 3, 1)).reshape(_B * _T, _C, _N)
    adjt = adj.T

    log2e = jnp.float32(np.log2(np.e))
    eye_h = jnp.eye(_H, dtype=jnp.float32)
    blk_src = jnp.einsum('hd,hg->ghd', a_src, eye_h).reshape(_H, _C)
    blk_dst = jnp.einsum('hd,hg->ghd', a_dst, eye_h).reshape(_H, _C)
    A = jnp.concatenate([blk_src, blk_dst], axis=0) * log2e

    cb_col = conv_b.reshape(_C, 1)

    y = pl.pallas_call(
        _attn_body,
        grid=(_B * _T // _S,),
        in_specs=[
            pl.BlockSpec((_S, _C, _N), lambda i: (i, 0, 0)),
            pl.BlockSpec((_N, _N), lambda i: (0, 0)),
            pl.BlockSpec((_C, _C), lambda i: (0, 0)),
            pl.BlockSpec((2 * _H, _C), lambda i: (0, 0)),
            pl.BlockSpec((_C, _C), lambda i: (0, 0)),
            pl.BlockSpec((_C, 1), lambda i: (0, 0)),
        ],
        out_specs=pl.BlockSpec((_S, _C, _N), lambda i: (i, 0, 0)),
        out_shape=jax.ShapeDtypeStruct((_B * _T, _C, _N), jnp.float32),
    )(xt, adjt, W.T, A, conv_w, cb_col)

    return jnp.transpose(y.reshape(_B, _T, _C, _N), (0, 3, 1, 2))
